# baseline (speedup 1.0000x reference)
import jax
import jax.numpy as jnp
from jax.experimental import pallas as pl

_TI = 500
_TO = 1000


def _knnconv_body(x_ref, w_ref, b_ref, o_ref):
    i = pl.program_id(0)
    ti, k, d = x_ref.shape
    x = x_ref[...].reshape(ti * k, d)
    h = jax.lax.dot_general(
        x, w_ref[...],
        dimension_numbers=(((1,), (1,)), ((), ())),
        preferred_element_type=jnp.float32,
    )
    h = h.reshape(ti, k, h.shape[-1])
    pooled = jnp.maximum(jnp.max(h, axis=1) + b_ref[...], 0.0)

    @pl.when(i % 2 == 0)
    def _():
        o_ref[0:_TI, :] = pooled

    @pl.when(i % 2 == 1)
    def _():
        o_ref[_TI:2 * _TI, :] = pooled


def kernel(agg_feat, W0, b0):
    n, k, d = agg_feat.shape
    o = W0.shape[0]
    grid = n // _TI
    b2 = b0.reshape(1, o)
    return pl.pallas_call(
        _knnconv_body,
        grid=(grid,),
        in_specs=[
            pl.BlockSpec((_TI, k, d), lambda i: (i, 0, 0)),
            pl.BlockSpec((o, d), lambda i: (0, 0)),
            pl.BlockSpec((1, o), lambda i: (0, 0)),
        ],
        out_specs=pl.BlockSpec((_TO, o), lambda i: (i // 2, 0)),
        out_shape=jax.ShapeDtypeStruct((n, o), jnp.float32),
    )(agg_feat, W0, b2)


# speedup vs baseline: 1.0097x; 1.0097x over previous
import jax
import jax.numpy as jnp
from jax.experimental import pallas as pl


def _knnconv_body(x_ref, w_ref, b_ref, o_ref):
    tn, k, d = x_ref.shape
    x = x_ref[...].reshape(tn * k, d)
    h = jax.lax.dot_general(
        x, w_ref[...],
        dimension_numbers=(((1,), (1,)), ((), ())),
        preferred_element_type=jnp.float32,
    )
    h = h.reshape(tn, k, h.shape[-1])
    pooled = jnp.max(h, axis=1) + b_ref[...]
    o_ref[...] = jnp.maximum(pooled, 0.0)


def kernel(agg_feat, W0, b0):
    n, k, d = agg_feat.shape
    o = W0.shape[0]
    tn = 1000
    grid = n // tn
    b2 = b0.reshape(1, o)
    return pl.pallas_call(
        _knnconv_body,
        grid=(grid,),
        in_specs=[
            pl.BlockSpec((tn, k, d), lambda i: (i, 0, 0)),
            pl.BlockSpec((o, d), lambda i: (0, 0)),
            pl.BlockSpec((1, o), lambda i: (0, 0)),
        ],
        out_specs=pl.BlockSpec((tn, o), lambda i: (i, 0)),
        out_shape=jax.ShapeDtypeStruct((n, o), jnp.float32),
    )(agg_feat, W0, b2)
